# baseline (speedup 1.0000x reference)
import functools

import jax
import jax.numpy as jnp
from jax import lax
from jax.experimental import pallas as pl
from jax.experimental.pallas import tpu as pltpu
from jax.experimental.pallas import tpu_sc as plsc

N = 10000
E = 320000
DN = 128
DE = 16
NC = 2
NS = 16
NW = NC * NS
NPB = N // NS
E8 = E * DE // 128

KA = 10
SCA = E // (KA * 128)
ITA = -(-SCA // NW)
KB = 2
CH = KB * 128
SCB = E // CH
ITB = -(-SCB // NS)
NSLOT = 3
DH = DN // NC


def _node_body(nf, wfc_t, ws_t, wd_t, zl_out, zr_out, ps_out, pd_out):
  z = jnp.dot(nf[...], wfc_t[...], preferred_element_type=jnp.float32)
  zl_out[...] = z[:, :DH]
  zr_out[...] = z[:, DH:]
  ps_out[...] = jnp.dot(z, ws_t[...], preferred_element_type=jnp.float32)
  pd_out[...] = jnp.dot(z, wd_t[...], preferred_element_type=jnp.float32)


def _edge_body(eft, we_t, bt, out):
  out[...] = lax.dot_general(
      eft[...], we_t[...], (((0,), (0,)), ((), ())),
      preferred_element_type=jnp.float32) + bt[...]


def _att_body(pre8, pre16, eye16, wcr, featt_out, ae_out):
  x8 = pre8[...]
  f8 = jnp.where(x8 >= 0.0, x8, 0.01 * x8)
  ae_out[...] = jnp.exp(
      jnp.dot(f8, wcr[...], preferred_element_type=jnp.float32))
  x16 = pre16[...]
  f16 = jnp.where(x16 >= 0.0, x16, 0.01 * x16)
  featt_out[...] = lax.dot_general(
      eye16[...], f16, (((1,), (1,)), ((), ())),
      preferred_element_type=jnp.float32)


def _final_body(p, d, h_out):
  den = jnp.maximum(d[...], 1e-16)
  h_out[...] = p[...] / den[:, 0:1]


def _pre_body(ps_hbm, pd_hbm, ep_hbm, ei_hbm, pre_hbm,
              idxs, idxd, buf, sem):
  c = lax.axis_index("c")
  s = lax.axis_index("s")
  w = s * NC + c

  def step(i, carry):
    sc = i * NW + w

    @pl.when(sc < SCA)
    def _():
      eoff = sc * (KA * 128)
      pltpu.sync_copy(ei_hbm.at[0, pl.ds(eoff, KA * 128)], idxs)
      pltpu.sync_copy(ei_hbm.at[1, pl.ds(eoff, KA * 128)], idxd)
      pltpu.sync_copy(ep_hbm.at[pl.ds(eoff, KA * 128)], buf)
      gs = [
          pltpu.async_copy(ps_hbm.at[idxs.at[pl.ds(j * 128, 128)]],
                           buf.at[pl.ds(j * 128, 128)], sem, add=True)
          for j in range(KA)
      ]
      for g in gs:
        g.wait()
      gd = [
          pltpu.async_copy(pd_hbm.at[idxd.at[pl.ds(j * 128, 128)]],
                           buf.at[pl.ds(j * 128, 128)], sem, add=True)
          for j in range(KA)
      ]
      for g in gd:
        g.wait()
      pltpu.sync_copy(buf, pre_hbm.at[pl.ds(eoff, KA * 128)])

    return carry

  lax.fori_loop(0, ITA, step, 0)


def _agg_body(zl_hbm, zr_hbm, ae_hbm, ei_hbm, zh_hbm, zd_hbm,
              hp_hbm, dp_hbm,
              idxs, idxd, rows, avr, avd, hsh, dsh, semL, semG, semS):
  c = lax.axis_index("c")
  s = lax.axis_index("s")

  pltpu.sync_copy(zh_hbm, hsh.at[pl.ds(s * NPB, NPB)])
  pltpu.sync_copy(zd_hbm, dsh.at[pl.ds(s * NPB, NPB)])
  plsc.subcore_barrier()


  def idx_load_descs(slot, eoff, semref):
    ds_ = [
        pltpu.make_async_copy(ei_hbm.at[0, pl.ds(eoff, CH)],
                              idxs.at[pl.ds(slot * CH, CH)], semref),
        pltpu.make_async_copy(ae_hbm.at[pl.ds(eoff, CH)],
                              avr.at[pl.ds(slot * CH, CH)], semref),
        pltpu.make_async_copy(ae_hbm.at[pl.ds(eoff, CH), pl.ds(0, DE // 2)],
                              avd.at[pl.ds(slot * CH, CH)], semref),
    ]
    ds_ += [
        pltpu.make_async_copy(ei_hbm.at[1, pl.ds(eoff + j * 128, 128)],
                              idxd.at[slot * KB + j], semref)
        for j in range(KB)
    ]
    return ds_

  def gather_descs(slot, zsrc, semref):
    return [
        pltpu.make_async_copy(
            zsrc.at[idxs.at[pl.ds(slot * CH + j * 128, 128)]],
            rows.at[pl.ds(slot * CH + j * 128, 128)], semref)
        for j in range(KB)
    ]

  def scat_h_descs(slot, semref):
    return [
        pltpu.make_async_copy(rows.at[pl.ds(slot * CH + j * 128, 128)],
                              hsh.at[idxd.at[slot * KB + j]], semref)
        for j in range(KB)
    ]

  def scat_d_descs(slot, semref):
    return [
        pltpu.make_async_copy(avd.at[pl.ds(slot * CH + j * 128, 128)],
                              dsh.at[idxd.at[slot * KB + j]], semref)
        for j in range(KB)
    ]

  def tick(t, r4):
    rD = (r4 - 3) % NSLOT
    scD = (t - 3) * NS + s

    @pl.when((t >= 3) & (scD < SCB))
    def _():
      for d in scat_h_descs(rD, semS.at[rD]):
        d.wait()

      @pl.when(c == 0)
      def _():
        for d in scat_d_descs(rD, semS.at[rD]):
          d.wait()

    scL = t * NS + s

    @pl.when(scL < SCB)
    def _():
      for d in idx_load_descs(r4, scL * CH, semL.at[r4]):
        d.start()

    rG = (r4 - 1) % NSLOT
    scG = (t - 1) * NS + s

    @pl.when((t >= 1) & (scG < SCB))
    def _():
      for d in idx_load_descs(rG, scG * CH, semL.at[rG]):
        d.wait()

      @pl.when(c == 0)
      def _():
        for d in gather_descs(rG, zl_hbm, semG.at[rG]):
          d.start()

      @pl.when(c == 1)
      def _():
        for d in gather_descs(rG, zr_hbm, semG.at[rG]):
          d.start()

    rC = (r4 - 2) % NSLOT
    scC = (t - 2) * NS + s

    @pl.when((t >= 2) & (scC < SCB))
    def _():
      @pl.when(c == 0)
      def _():
        for d in gather_descs(rC, zl_hbm, semG.at[rC]):
          d.wait()

      @pl.when(c == 1)
      def _():
        for d in gather_descs(rC, zr_hbm, semG.at[rC]):
          d.wait()

      def scale_grp(g, carry2):
        for l in range(16):
          r = rC * CH + g * 16 + l
          va = avr[r]
          for k in range(DH // 16):
            sl = pl.ds(k * 16, 16)
            rows[r, sl] = rows[r, sl] * va
        return carry2

      lax.fori_loop(0, KB * 8, scale_grp, 0)

      for d in scat_h_descs(rC, semS.at[rC]):
        d.start(add=True)

      @pl.when(c == 0)
      def _():
        for d in scat_d_descs(rC, semS.at[rC]):
          d.start(add=True)

  def tickset(i, carry):
    for r4 in range(NSLOT):
      tick(i * NSLOT + r4, r4)
    return carry

  nticks = ITB + 3
  lax.fori_loop(0, -(-nticks // NSLOT), tickset, 0)
  plsc.subcore_barrier()

  @pl.when(c == 0)
  def _():
    pltpu.sync_copy(hsh.at[pl.ds(s * NPB, NPB)],
                    hp_hbm.at[pl.ds(s * NPB, NPB), pl.ds(0, DH)])
    pltpu.sync_copy(dsh.at[pl.ds(s * NPB, NPB)],
                    dp_hbm.at[pl.ds(s * NPB, NPB)])

  @pl.when(c == 1)
  def _():
    pltpu.sync_copy(hsh.at[pl.ds(s * NPB, NPB)],
                    hp_hbm.at[pl.ds(s * NPB, NPB), pl.ds(DH, DH)])


@functools.lru_cache(maxsize=None)
def _sc_kernels():
  mesh = plsc.VectorSubcoreMesh(
      core_axis_name="c", subcore_axis_name="s", num_cores=NC, num_subcores=NS
  )
  params = pltpu.CompilerParams(use_tc_tiling_on_sc=False)
  pre_sc = pl.kernel(
      _pre_body,
      out_type=jax.ShapeDtypeStruct((E, DE), jnp.float32),
      mesh=mesh,
      compiler_params=params,
      scratch_types=[
          pltpu.VMEM((KA * 128,), jnp.int32),
          pltpu.VMEM((KA * 128,), jnp.int32),
          pltpu.VMEM((KA * 128, DE), jnp.float32),
          pltpu.SemaphoreType.DMA,
      ],
  )
  agg_sc = pl.kernel(
      _agg_body,
      out_type=(
          jax.ShapeDtypeStruct((N, DN), jnp.float32),
          jax.ShapeDtypeStruct((N, DE // 2), jnp.float32),
      ),
      mesh=mesh,
      compiler_params=params,
      scratch_types=[
          pltpu.VMEM((NSLOT * CH,), jnp.int32),
          pltpu.VMEM((NSLOT * KB, 128), jnp.int32),
          pltpu.VMEM((NSLOT * CH, DH), jnp.float32),
          pltpu.VMEM((NSLOT * CH, DE), jnp.float32),
          pltpu.VMEM((NSLOT * CH, DE // 2), jnp.float32),
          pltpu.VMEM_SHARED((N, DH), jnp.float32),
          pltpu.VMEM_SHARED((N, DE // 2), jnp.float32),
          pltpu.SemaphoreType.DMA((NSLOT,)),
          pltpu.SemaphoreType.DMA((NSLOT,)),
          pltpu.SemaphoreType.DMA((NSLOT,)),
      ],
  )
  return pre_sc, agg_sc


def _tc_node(nfeats, wfc_t, ws_t, wd_t):
  blk = 1000
  return pl.pallas_call(
      _node_body,
      grid=(N // blk,),
      in_specs=[
          pl.BlockSpec((blk, DN), lambda i: (i, 0)),
          pl.BlockSpec((DN, DN), lambda i: (0, 0)),
          pl.BlockSpec((DN, DE), lambda i: (0, 0)),
          pl.BlockSpec((DN, DE), lambda i: (0, 0)),
      ],
      out_specs=[
          pl.BlockSpec((blk, DH), lambda i: (i, 0)),
          pl.BlockSpec((blk, DH), lambda i: (i, 0)),
          pl.BlockSpec((blk, DE), lambda i: (i, 0)),
          pl.BlockSpec((blk, DE), lambda i: (i, 0)),
      ],
      out_shape=[
          jax.ShapeDtypeStruct((N, DH), jnp.float32),
          jax.ShapeDtypeStruct((N, DH), jnp.float32),
          jax.ShapeDtypeStruct((N, DE), jnp.float32),
          jax.ShapeDtypeStruct((N, DE), jnp.float32),
      ],
  )(nfeats, wfc_t, ws_t, wd_t)


def _tc_edge(eft, we_t, bt):
  blk = E // 10
  return pl.pallas_call(
      _edge_body,
      grid=(10,),
      in_specs=[
          pl.BlockSpec((DE, blk), lambda i: (0, i)),
          pl.BlockSpec((DE, DE), lambda i: (0, 0)),
          pl.BlockSpec((1, DE), lambda i: (0, 0)),
      ],
      out_specs=pl.BlockSpec((blk, DE), lambda i: (i, 0)),
      out_shape=jax.ShapeDtypeStruct((E, DE), jnp.float32),
  )(eft, we_t, bt)


def _tc_att(pre8, pre16, eye16, wcr):
  blk8 = E8 // 10
  blk = E // 10
  return pl.pallas_call(
      _att_body,
      grid=(10,),
      in_specs=[
          pl.BlockSpec((blk8, 128), lambda i: (i, 0)),
          pl.BlockSpec((blk, DE), lambda i: (i, 0)),
          pl.BlockSpec((DE, DE), lambda i: (0, 0)),
          pl.BlockSpec((128, 128), lambda i: (0, 0)),
      ],
      out_specs=[
          pl.BlockSpec((DE, blk), lambda i: (0, i)),
          pl.BlockSpec((blk8, 128), lambda i: (i, 0)),
      ],
      out_shape=[
          jax.ShapeDtypeStruct((DE, E), jnp.float32),
          jax.ShapeDtypeStruct((E8, 128), jnp.float32),
      ],
  )(pre8, pre16, eye16, wcr)


def _tc_final(hp, dp):
  blk = 1000
  return pl.pallas_call(
      _final_body,
      grid=(N // blk,),
      in_specs=[
          pl.BlockSpec((blk, DN), lambda i: (i, 0)),
          pl.BlockSpec((blk, DE // 2), lambda i: (i, 0)),
      ],
      out_specs=pl.BlockSpec((blk, DN), lambda i: (i, 0)),
      out_shape=jax.ShapeDtypeStruct((N, DN), jnp.float32),
  )(hp, dp)


def kernel(nfeats, edge_index, efeats, W_fc, W_edge, b_edge, W_coef):
  ei = edge_index.astype(jnp.int32)

  wfc_t = W_fc.T
  ws_t = W_edge[:, :DN].T
  wd_t = W_edge[:, DN + DE:].T
  we_t = W_edge[:, DN:DN + DE].T
  eye8 = jnp.eye(8, dtype=jnp.float32)
  bt = b_edge[None, :]
  wcr = jnp.kron(eye8, W_coef[0][:, None] * jnp.ones((1, 16), jnp.float32))

  zl, zr, ps, pd = _tc_node(nfeats, wfc_t, ws_t, wd_t)

  eproj = _tc_edge(efeats.T, we_t, bt)

  pre_sc, agg_sc = _sc_kernels()
  pre = pre_sc(ps, pd, eproj, ei)

  eye16 = jnp.eye(DE, dtype=jnp.float32)
  featt, ae8 = _tc_att(pre.reshape(E8, 128), pre, eye16, wcr)
  feat_e = featt.T
  ae2 = ae8.reshape(E, DE)

  zh = jnp.zeros((NPB, DH), jnp.float32)
  zd = jnp.zeros((NPB, DE // 2), jnp.float32)
  hp, dp = agg_sc(zl, zr, ae2, ei, zh, zd)

  h = _tc_final(hp, dp)
  return (h, feat_e)


# speedup vs baseline: 1.1822x; 1.1822x over previous
import functools

import jax
import jax.numpy as jnp
from jax import lax
from jax.experimental import pallas as pl
from jax.experimental.pallas import tpu as pltpu
from jax.experimental.pallas import tpu_sc as plsc

N = 10000
E = 320000
DN = 128
DE = 16
NC = 2
NS = 16
NW = NC * NS
NPB = N // NS
E8 = E * DE // 128

KA = 10
SCA = E // (KA * 128)
ITA = -(-SCA // NW)
KB = 2
CH = KB * 128
SCB = E // CH
ITB = -(-SCB // NS)
NSLOT = 3
DH = DN // NC


def _node_body(nf, wfc_t, ws_t, wd_t, zl_out, zr_out, ps_out, pd_out):
  z = jnp.dot(nf[...], wfc_t[...], preferred_element_type=jnp.float32)
  zl_out[...] = z[:, :DH]
  zr_out[...] = z[:, DH:]
  ps_out[...] = jnp.dot(z, ws_t[...], preferred_element_type=jnp.float32)
  pd_out[...] = jnp.dot(z, wd_t[...], preferred_element_type=jnp.float32)


def _edge_body(eft, we_t, bt, out):
  out[...] = lax.dot_general(
      eft[...], we_t[...], (((0,), (0,)), ((), ())),
      preferred_element_type=jnp.float32) + bt[...]


def _att_body(pre8, wcr, feat_out, ae_out):
  x = pre8[...]
  f = jnp.where(x >= 0.0, x, 0.01 * x)
  feat_out[...] = f
  ae_out[...] = jnp.exp(
      jnp.dot(f, wcr[...], preferred_element_type=jnp.float32))


def _final_body(p, d, h_out):
  den = jnp.maximum(d[...], 1e-16)
  h_out[...] = p[...] / den[:, 0:1]


def _pre_body(ps_hbm, pd_hbm, ep_hbm, ei_hbm, pre_hbm,
              idxs, idxd, buf, sem):
  c = lax.axis_index("c")
  s = lax.axis_index("s")
  w = s * NC + c

  def step(i, carry):
    sc = i * NW + w

    @pl.when(sc < SCA)
    def _():
      eoff = sc * (KA * 128)
      pltpu.sync_copy(ei_hbm.at[0, pl.ds(eoff, KA * 128)], idxs)
      pltpu.sync_copy(ei_hbm.at[1, pl.ds(eoff, KA * 128)], idxd)
      pltpu.sync_copy(ep_hbm.at[pl.ds(eoff, KA * 128)], buf)
      gs = [
          pltpu.async_copy(ps_hbm.at[idxs.at[pl.ds(j * 128, 128)]],
                           buf.at[pl.ds(j * 128, 128)], sem, add=True)
          for j in range(KA)
      ]
      for g in gs:
        g.wait()
      gd = [
          pltpu.async_copy(pd_hbm.at[idxd.at[pl.ds(j * 128, 128)]],
                           buf.at[pl.ds(j * 128, 128)], sem, add=True)
          for j in range(KA)
      ]
      for g in gd:
        g.wait()
      pltpu.sync_copy(buf, pre_hbm.at[pl.ds(eoff, KA * 128)])

    return carry

  lax.fori_loop(0, ITA, step, 0)


def _agg_body(zl_hbm, zr_hbm, ae_hbm, ei_hbm, zh_hbm, zd_hbm,
              hp_hbm, dp_hbm,
              idxs, idxd, rows, avr, avd, hsh, dsh, semL, semG, semS):
  c = lax.axis_index("c")
  s = lax.axis_index("s")

  pltpu.sync_copy(zh_hbm, hsh.at[pl.ds(s * NPB, NPB)])
  pltpu.sync_copy(zd_hbm, dsh.at[pl.ds(s * NPB, NPB)])
  plsc.subcore_barrier()


  def idx_load_descs(slot, eoff, semref):
    ds_ = [
        pltpu.make_async_copy(ei_hbm.at[0, pl.ds(eoff, CH)],
                              idxs.at[pl.ds(slot * CH, CH)], semref),
        pltpu.make_async_copy(ae_hbm.at[pl.ds(eoff, CH)],
                              avr.at[pl.ds(slot * CH, CH)], semref),
        pltpu.make_async_copy(ae_hbm.at[pl.ds(eoff, CH), pl.ds(0, DE // 2)],
                              avd.at[pl.ds(slot * CH, CH)], semref),
    ]
    ds_ += [
        pltpu.make_async_copy(ei_hbm.at[1, pl.ds(eoff + j * 128, 128)],
                              idxd.at[slot * KB + j], semref)
        for j in range(KB)
    ]
    return ds_

  def gather_descs(slot, zsrc, semref):
    return [
        pltpu.make_async_copy(
            zsrc.at[idxs.at[pl.ds(slot * CH + j * 128, 128)]],
            rows.at[pl.ds(slot * CH + j * 128, 128)], semref)
        for j in range(KB)
    ]

  def scat_h_descs(slot, semref):
    return [
        pltpu.make_async_copy(rows.at[pl.ds(slot * CH + j * 128, 128)],
                              hsh.at[idxd.at[slot * KB + j]], semref)
        for j in range(KB)
    ]

  def scat_d_descs(slot, semref):
    return [
        pltpu.make_async_copy(avd.at[pl.ds(slot * CH + j * 128, 128)],
                              dsh.at[idxd.at[slot * KB + j]], semref)
        for j in range(KB)
    ]

  def tick(t, r4):
    rD = (r4 - 3) % NSLOT
    scD = (t - 3) * NS + s

    @pl.when((t >= 3) & (scD < SCB))
    def _():
      for d in scat_h_descs(rD, semS.at[rD]):
        d.wait()

      @pl.when(c == 0)
      def _():
        for d in scat_d_descs(rD, semS.at[rD]):
          d.wait()

    scL = t * NS + s

    @pl.when(scL < SCB)
    def _():
      for d in idx_load_descs(r4, scL * CH, semL.at[r4]):
        d.start()

    rG = (r4 - 1) % NSLOT
    scG = (t - 1) * NS + s

    @pl.when((t >= 1) & (scG < SCB))
    def _():
      for d in idx_load_descs(rG, scG * CH, semL.at[rG]):
        d.wait()

      @pl.when(c == 0)
      def _():
        for d in gather_descs(rG, zl_hbm, semG.at[rG]):
          d.start()

      @pl.when(c == 1)
      def _():
        for d in gather_descs(rG, zr_hbm, semG.at[rG]):
          d.start()

    rC = (r4 - 2) % NSLOT
    scC = (t - 2) * NS + s

    @pl.when((t >= 2) & (scC < SCB))
    def _():
      @pl.when(c == 0)
      def _():
        for d in gather_descs(rC, zl_hbm, semG.at[rC]):
          d.wait()

      @pl.when(c == 1)
      def _():
        for d in gather_descs(rC, zr_hbm, semG.at[rC]):
          d.wait()

      def scale_grp(g, carry2):
        for l in range(16):
          r = rC * CH + g * 16 + l
          va = avr[r]
          for k in range(DH // 16):
            sl = pl.ds(k * 16, 16)
            rows[r, sl] = rows[r, sl] * va
        return carry2

      lax.fori_loop(0, KB * 8, scale_grp, 0)

      for d in scat_h_descs(rC, semS.at[rC]):
        d.start(add=True)

      @pl.when(c == 0)
      def _():
        for d in scat_d_descs(rC, semS.at[rC]):
          d.start(add=True)

  def tickset(i, carry):
    for r4 in range(NSLOT):
      tick(i * NSLOT + r4, r4)
    return carry

  nticks = ITB + 3
  lax.fori_loop(0, -(-nticks // NSLOT), tickset, 0)
  plsc.subcore_barrier()

  @pl.when(c == 0)
  def _():
    pltpu.sync_copy(hsh.at[pl.ds(s * NPB, NPB)],
                    hp_hbm.at[pl.ds(s * NPB, NPB), pl.ds(0, DH)])
    pltpu.sync_copy(dsh.at[pl.ds(s * NPB, NPB)],
                    dp_hbm.at[pl.ds(s * NPB, NPB)])

  @pl.when(c == 1)
  def _():
    pltpu.sync_copy(hsh.at[pl.ds(s * NPB, NPB)],
                    hp_hbm.at[pl.ds(s * NPB, NPB), pl.ds(DH, DH)])


@functools.lru_cache(maxsize=None)
def _sc_kernels():
  mesh = plsc.VectorSubcoreMesh(
      core_axis_name="c", subcore_axis_name="s", num_cores=NC, num_subcores=NS
  )
  params = pltpu.CompilerParams(use_tc_tiling_on_sc=False)
  pre_sc = pl.kernel(
      _pre_body,
      out_type=jax.ShapeDtypeStruct((E, DE), jnp.float32),
      mesh=mesh,
      compiler_params=params,
      scratch_types=[
          pltpu.VMEM((KA * 128,), jnp.int32),
          pltpu.VMEM((KA * 128,), jnp.int32),
          pltpu.VMEM((KA * 128, DE), jnp.float32),
          pltpu.SemaphoreType.DMA,
      ],
  )
  agg_sc = pl.kernel(
      _agg_body,
      out_type=(
          jax.ShapeDtypeStruct((N, DN), jnp.float32),
          jax.ShapeDtypeStruct((N, DE // 2), jnp.float32),
      ),
      mesh=mesh,
      compiler_params=params,
      scratch_types=[
          pltpu.VMEM((NSLOT * CH,), jnp.int32),
          pltpu.VMEM((NSLOT * KB, 128), jnp.int32),
          pltpu.VMEM((NSLOT * CH, DH), jnp.float32),
          pltpu.VMEM((NSLOT * CH, DE), jnp.float32),
          pltpu.VMEM((NSLOT * CH, DE // 2), jnp.float32),
          pltpu.VMEM_SHARED((N, DH), jnp.float32),
          pltpu.VMEM_SHARED((N, DE // 2), jnp.float32),
          pltpu.SemaphoreType.DMA((NSLOT,)),
          pltpu.SemaphoreType.DMA((NSLOT,)),
          pltpu.SemaphoreType.DMA((NSLOT,)),
      ],
  )
  return pre_sc, agg_sc


def _tc_node(nfeats, wfc_t, ws_t, wd_t):
  blk = 1000
  return pl.pallas_call(
      _node_body,
      grid=(N // blk,),
      in_specs=[
          pl.BlockSpec((blk, DN), lambda i: (i, 0)),
          pl.BlockSpec((DN, DN), lambda i: (0, 0)),
          pl.BlockSpec((DN, DE), lambda i: (0, 0)),
          pl.BlockSpec((DN, DE), lambda i: (0, 0)),
      ],
      out_specs=[
          pl.BlockSpec((blk, DH), lambda i: (i, 0)),
          pl.BlockSpec((blk, DH), lambda i: (i, 0)),
          pl.BlockSpec((blk, DE), lambda i: (i, 0)),
          pl.BlockSpec((blk, DE), lambda i: (i, 0)),
      ],
      out_shape=[
          jax.ShapeDtypeStruct((N, DH), jnp.float32),
          jax.ShapeDtypeStruct((N, DH), jnp.float32),
          jax.ShapeDtypeStruct((N, DE), jnp.float32),
          jax.ShapeDtypeStruct((N, DE), jnp.float32),
      ],
  )(nfeats, wfc_t, ws_t, wd_t)


def _tc_edge(eft, we_t, bt):
  blk = E // 10
  return pl.pallas_call(
      _edge_body,
      grid=(10,),
      in_specs=[
          pl.BlockSpec((DE, blk), lambda i: (0, i)),
          pl.BlockSpec((DE, DE), lambda i: (0, 0)),
          pl.BlockSpec((1, DE), lambda i: (0, 0)),
      ],
      out_specs=pl.BlockSpec((blk, DE), lambda i: (i, 0)),
      out_shape=jax.ShapeDtypeStruct((E, DE), jnp.float32),
  )(eft, we_t, bt)


def _tc_att(pre8, wcr):
  blk8 = E8 // 10
  return pl.pallas_call(
      _att_body,
      grid=(10,),
      in_specs=[
          pl.BlockSpec((blk8, 128), lambda i: (i, 0)),
          pl.BlockSpec((128, 128), lambda i: (0, 0)),
      ],
      out_specs=[
          pl.BlockSpec((blk8, 128), lambda i: (i, 0)),
          pl.BlockSpec((blk8, 128), lambda i: (i, 0)),
      ],
      out_shape=[
          jax.ShapeDtypeStruct((E8, 128), jnp.float32),
          jax.ShapeDtypeStruct((E8, 128), jnp.float32),
      ],
  )(pre8, wcr)


def _tc_final(hp, dp):
  blk = 1000
  return pl.pallas_call(
      _final_body,
      grid=(N // blk,),
      in_specs=[
          pl.BlockSpec((blk, DN), lambda i: (i, 0)),
          pl.BlockSpec((blk, DE // 2), lambda i: (i, 0)),
      ],
      out_specs=pl.BlockSpec((blk, DN), lambda i: (i, 0)),
      out_shape=jax.ShapeDtypeStruct((N, DN), jnp.float32),
  )(hp, dp)


def kernel(nfeats, edge_index, efeats, W_fc, W_edge, b_edge, W_coef):
  ei = edge_index.astype(jnp.int32)

  wfc_t = W_fc.T
  ws_t = W_edge[:, :DN].T
  wd_t = W_edge[:, DN + DE:].T
  we_t = W_edge[:, DN:DN + DE].T
  eye8 = jnp.eye(8, dtype=jnp.float32)
  bt = b_edge[None, :]
  wcr = jnp.kron(eye8, W_coef[0][:, None] * jnp.ones((1, 16), jnp.float32))

  zl, zr, ps, pd = _tc_node(nfeats, wfc_t, ws_t, wd_t)

  eproj = _tc_edge(efeats.T, we_t, bt)

  pre_sc, agg_sc = _sc_kernels()
  pre = pre_sc(ps, pd, eproj, ei)

  feat8, ae8 = _tc_att(pre.reshape(E8, 128), wcr)
  feat_e = feat8.reshape(E, DE)
  ae2 = ae8.reshape(E, DE)

  zh = jnp.zeros((NPB, DH), jnp.float32)
  zd = jnp.zeros((NPB, DE // 2), jnp.float32)
  hp, dp = agg_sc(zl, zr, ae2, ei, zh, zd)

  h = _tc_final(hp, dp)
  return (h, feat_e)


# speedup vs baseline: 1.3116x; 1.1095x over previous
import functools

import jax
import jax.numpy as jnp
from jax import lax
from jax.experimental import pallas as pl
from jax.experimental.pallas import tpu as pltpu
from jax.experimental.pallas import tpu_sc as plsc

N = 10000
E = 320000
DN = 128
DE = 16
NC = 2
NS = 16
NW = NC * NS
NPB = N // NS
E8 = E * DE // 128

KA = 5
KAC = KA * 128
SCA = E // KAC
ITA = -(-SCA // NW)
NSLOTA = 5
KB = 2
CH = KB * 128
SCB = E // CH
ITB = -(-SCB // NS)
NSLOT = 3
DH = DN // NC


def _node_body(nf, wfc_t, ws_t, wd_t, zl_out, zr_out, ps_out, pd_out):
  z = jnp.dot(nf[...], wfc_t[...], preferred_element_type=jnp.float32)
  zl_out[...] = z[:, :DH]
  zr_out[...] = z[:, DH:]
  ps_out[...] = jnp.dot(z, ws_t[...], preferred_element_type=jnp.float32)
  pd_out[...] = jnp.dot(z, wd_t[...], preferred_element_type=jnp.float32)


def _edge_body(ef8, wbd, bt, out):
  out[...] = (
      jnp.dot(ef8[...], wbd[...], preferred_element_type=jnp.float32) + bt[...]
  )


def _att_body(pre8, wcr, feat_out, ae_out):
  x = pre8[...]
  f = jnp.where(x >= 0.0, x, 0.01 * x)
  feat_out[...] = f
  ae_out[...] = jnp.exp(
      jnp.dot(f, wcr[...], preferred_element_type=jnp.float32))


def _final_body(p, d, h_out):
  den = jnp.maximum(d[...], 1e-16)
  h_out[...] = p[...] / den[:, 0:1]


def _pre_body(ps_hbm, pd_hbm, ep_hbm, ei_hbm, pre_hbm,
              idxs, idxd, buf, semL, semA, semB, semW):
  c = lax.axis_index("c")
  s = lax.axis_index("s")
  w = s * NC + c


  def load_descs(slot, eoff, semref):
    return [
        pltpu.make_async_copy(ei_hbm.at[0, pl.ds(eoff, KAC)],
                              idxs.at[pl.ds(slot * KAC, KAC)], semref),
        pltpu.make_async_copy(ei_hbm.at[1, pl.ds(eoff, KAC)],
                              idxd.at[pl.ds(slot * KAC, KAC)], semref),
        pltpu.make_async_copy(ep_hbm.at[pl.ds(eoff, KAC)],
                              buf.at[pl.ds(slot * KAC, KAC)], semref),
    ]

  def gadd_descs(slot, src_hbm, idxref, semref):
    return [
        pltpu.make_async_copy(
            src_hbm.at[idxref.at[pl.ds(slot * KAC + j * 128, 128)]],
            buf.at[pl.ds(slot * KAC + j * 128, 128)], semref)
        for j in range(KA)
    ]

  def wr_desc(slot, eoff, semref):
    return pltpu.make_async_copy(buf.at[pl.ds(slot * KAC, KAC)],
                                 pre_hbm.at[pl.ds(eoff, KAC)], semref)

  def tick(t, r5):
    rW = (r5 - 4) % NSLOTA
    scW = (t - 4) * NW + w

    @pl.when((t >= 4) & (scW < SCA))
    def _():
      wr_desc(rW, scW * KAC, semW.at[rW]).wait()

    scL = t * NW + w

    @pl.when(scL < SCA)
    def _():
      for d in load_descs(r5, scL * KAC, semL.at[r5]):
        d.start()

    rA = (r5 - 1) % NSLOTA
    scA = (t - 1) * NW + w

    @pl.when((t >= 1) & (scA < SCA))
    def _():
      for d in load_descs(rA, scA * KAC, semL.at[rA]):
        d.wait()
      for d in gadd_descs(rA, ps_hbm, idxs, semA.at[rA]):
        d.start(add=True)

    rB = (r5 - 2) % NSLOTA
    scB = (t - 2) * NW + w

    @pl.when((t >= 2) & (scB < SCA))
    def _():
      for d in gadd_descs(rB, ps_hbm, idxs, semA.at[rB]):
        d.wait()
      for d in gadd_descs(rB, pd_hbm, idxd, semB.at[rB]):
        d.start(add=True)

    rC = (r5 - 3) % NSLOTA
    scC = (t - 3) * NW + w

    @pl.when((t >= 3) & (scC < SCA))
    def _():
      for d in gadd_descs(rC, pd_hbm, idxd, semB.at[rC]):
        d.wait()
      wr_desc(rC, scC * KAC, semW.at[rC]).start()

  def tickset(i, carry):
    for r5 in range(NSLOTA):
      tick(i * NSLOTA + r5, r5)
    return carry

  nticks = ITA + 4
  lax.fori_loop(0, -(-nticks // NSLOTA), tickset, 0)


def _agg_body(zl_hbm, zr_hbm, ae_hbm, ei_hbm, zh_hbm, zd_hbm,
              hp_hbm, dp_hbm,
              idxs, idxd, rows, avr, avd, hsh, dsh, semL, semG, semS):
  c = lax.axis_index("c")
  s = lax.axis_index("s")

  pltpu.sync_copy(zh_hbm, hsh.at[pl.ds(s * NPB, NPB)])
  pltpu.sync_copy(zd_hbm, dsh.at[pl.ds(s * NPB, NPB)])
  plsc.subcore_barrier()


  def idx_load_descs(slot, eoff, semref):
    ds_ = [
        pltpu.make_async_copy(ei_hbm.at[0, pl.ds(eoff, CH)],
                              idxs.at[pl.ds(slot * CH, CH)], semref),
        pltpu.make_async_copy(ae_hbm.at[pl.ds(eoff, CH)],
                              avr.at[pl.ds(slot * CH, CH)], semref),
        pltpu.make_async_copy(ae_hbm.at[pl.ds(eoff, CH), pl.ds(0, DE // 2)],
                              avd.at[pl.ds(slot * CH, CH)], semref),
    ]
    ds_ += [
        pltpu.make_async_copy(ei_hbm.at[1, pl.ds(eoff + j * 128, 128)],
                              idxd.at[slot * KB + j], semref)
        for j in range(KB)
    ]
    return ds_

  def gather_descs(slot, zsrc, semref):
    return [
        pltpu.make_async_copy(
            zsrc.at[idxs.at[pl.ds(slot * CH + j * 128, 128)]],
            rows.at[pl.ds(slot * CH + j * 128, 128)], semref)
        for j in range(KB)
    ]

  def scat_h_descs(slot, semref):
    return [
        pltpu.make_async_copy(rows.at[pl.ds(slot * CH + j * 128, 128)],
                              hsh.at[idxd.at[slot * KB + j]], semref)
        for j in range(KB)
    ]

  def scat_d_descs(slot, semref):
    return [
        pltpu.make_async_copy(avd.at[pl.ds(slot * CH + j * 128, 128)],
                              dsh.at[idxd.at[slot * KB + j]], semref)
        for j in range(KB)
    ]

  def tick(t, r4):
    rD = (r4 - 3) % NSLOT
    scD = (t - 3) * NS + s

    @pl.when((t >= 3) & (scD < SCB))
    def _():
      for d in scat_h_descs(rD, semS.at[rD]):
        d.wait()

      @pl.when(c == 0)
      def _():
        for d in scat_d_descs(rD, semS.at[rD]):
          d.wait()

    scL = t * NS + s

    @pl.when(scL < SCB)
    def _():
      for d in idx_load_descs(r4, scL * CH, semL.at[r4]):
        d.start()

    rG = (r4 - 1) % NSLOT
    scG = (t - 1) * NS + s

    @pl.when((t >= 1) & (scG < SCB))
    def _():
      for d in idx_load_descs(rG, scG * CH, semL.at[rG]):
        d.wait()

      @pl.when(c == 0)
      def _():
        for d in gather_descs(rG, zl_hbm, semG.at[rG]):
          d.start()

      @pl.when(c == 1)
      def _():
        for d in gather_descs(rG, zr_hbm, semG.at[rG]):
          d.start()

    rC = (r4 - 2) % NSLOT
    scC = (t - 2) * NS + s

    @pl.when((t >= 2) & (scC < SCB))
    def _():
      @pl.when(c == 0)
      def _():
        for d in gather_descs(rC, zl_hbm, semG.at[rC]):
          d.wait()

      @pl.when(c == 1)
      def _():
        for d in gather_descs(rC, zr_hbm, semG.at[rC]):
          d.wait()

      def scale_grp(g, carry2):
        for l in range(16):
          r = rC * CH + g * 16 + l
          va = avr[r]
          for k in range(DH // 16):
            sl = pl.ds(k * 16, 16)
            rows[r, sl] = rows[r, sl] * va
        return carry2

      lax.fori_loop(0, KB * 8, scale_grp, 0)

      for d in scat_h_descs(rC, semS.at[rC]):
        d.start(add=True)

      @pl.when(c == 0)
      def _():
        for d in scat_d_descs(rC, semS.at[rC]):
          d.start(add=True)

  def tickset(i, carry):
    for r4 in range(NSLOT):
      tick(i * NSLOT + r4, r4)
    return carry

  nticks = ITB + 3
  lax.fori_loop(0, -(-nticks // NSLOT), tickset, 0)
  plsc.subcore_barrier()

  @pl.when(c == 0)
  def _():
    pltpu.sync_copy(hsh.at[pl.ds(s * NPB, NPB)],
                    hp_hbm.at[pl.ds(s * NPB, NPB), pl.ds(0, DH)])
    pltpu.sync_copy(dsh.at[pl.ds(s * NPB, NPB)],
                    dp_hbm.at[pl.ds(s * NPB, NPB)])

  @pl.when(c == 1)
  def _():
    pltpu.sync_copy(hsh.at[pl.ds(s * NPB, NPB)],
                    hp_hbm.at[pl.ds(s * NPB, NPB), pl.ds(DH, DH)])


@functools.lru_cache(maxsize=None)
def _sc_kernels():
  mesh = plsc.VectorSubcoreMesh(
      core_axis_name="c", subcore_axis_name="s", num_cores=NC, num_subcores=NS
  )
  params = pltpu.CompilerParams(use_tc_tiling_on_sc=False)
  pre_sc = pl.kernel(
      _pre_body,
      out_type=jax.ShapeDtypeStruct((E, DE), jnp.float32),
      mesh=mesh,
      compiler_params=params,
      scratch_types=[
          pltpu.VMEM((NSLOTA * KAC,), jnp.int32),
          pltpu.VMEM((NSLOTA * KAC,), jnp.int32),
          pltpu.VMEM((NSLOTA * KAC, DE), jnp.float32),
          pltpu.SemaphoreType.DMA((NSLOTA,)),
          pltpu.SemaphoreType.DMA((NSLOTA,)),
          pltpu.SemaphoreType.DMA((NSLOTA,)),
          pltpu.SemaphoreType.DMA((NSLOTA,)),
      ],
  )
  agg_sc = pl.kernel(
      _agg_body,
      out_type=(
          jax.ShapeDtypeStruct((N, DN), jnp.float32),
          jax.ShapeDtypeStruct((N, DE // 2), jnp.float32),
      ),
      mesh=mesh,
      compiler_params=params,
      scratch_types=[
          pltpu.VMEM((NSLOT * CH,), jnp.int32),
          pltpu.VMEM((NSLOT * KB, 128), jnp.int32),
          pltpu.VMEM((NSLOT * CH, DH), jnp.float32),
          pltpu.VMEM((NSLOT * CH, DE), jnp.float32),
          pltpu.VMEM((NSLOT * CH, DE // 2), jnp.float32),
          pltpu.VMEM_SHARED((N, DH), jnp.float32),
          pltpu.VMEM_SHARED((N, DE // 2), jnp.float32),
          pltpu.SemaphoreType.DMA((NSLOT,)),
          pltpu.SemaphoreType.DMA((NSLOT,)),
          pltpu.SemaphoreType.DMA((NSLOT,)),
      ],
  )
  return pre_sc, agg_sc


def _tc_node(nfeats, wfc_t, ws_t, wd_t):
  blk = 1000
  return pl.pallas_call(
      _node_body,
      grid=(N // blk,),
      in_specs=[
          pl.BlockSpec((blk, DN), lambda i: (i, 0)),
          pl.BlockSpec((DN, DN), lambda i: (0, 0)),
          pl.BlockSpec((DN, DE), lambda i: (0, 0)),
          pl.BlockSpec((DN, DE), lambda i: (0, 0)),
      ],
      out_specs=[
          pl.BlockSpec((blk, DH), lambda i: (i, 0)),
          pl.BlockSpec((blk, DH), lambda i: (i, 0)),
          pl.BlockSpec((blk, DE), lambda i: (i, 0)),
          pl.BlockSpec((blk, DE), lambda i: (i, 0)),
      ],
      out_shape=[
          jax.ShapeDtypeStruct((N, DH), jnp.float32),
          jax.ShapeDtypeStruct((N, DH), jnp.float32),
          jax.ShapeDtypeStruct((N, DE), jnp.float32),
          jax.ShapeDtypeStruct((N, DE), jnp.float32),
      ],
  )(nfeats, wfc_t, ws_t, wd_t)


def _tc_edge(ef8, wbd, bt):
  blk = E8 // 10
  return pl.pallas_call(
      _edge_body,
      grid=(10,),
      in_specs=[
          pl.BlockSpec((blk, 128), lambda i: (i, 0)),
          pl.BlockSpec((128, 128), lambda i: (0, 0)),
          pl.BlockSpec((1, 128), lambda i: (0, 0)),
      ],
      out_specs=pl.BlockSpec((blk, 128), lambda i: (i, 0)),
      out_shape=jax.ShapeDtypeStruct((E8, 128), jnp.float32),
  )(ef8, wbd, bt)


def _tc_att(pre8, wcr):
  blk8 = E8 // 10
  return pl.pallas_call(
      _att_body,
      grid=(10,),
      in_specs=[
          pl.BlockSpec((blk8, 128), lambda i: (i, 0)),
          pl.BlockSpec((128, 128), lambda i: (0, 0)),
      ],
      out_specs=[
          pl.BlockSpec((blk8, 128), lambda i: (i, 0)),
          pl.BlockSpec((blk8, 128), lambda i: (i, 0)),
      ],
      out_shape=[
          jax.ShapeDtypeStruct((E8, 128), jnp.float32),
          jax.ShapeDtypeStruct((E8, 128), jnp.float32),
      ],
  )(pre8, wcr)


def _tc_final(hp, dp):
  blk = 1000
  return pl.pallas_call(
      _final_body,
      grid=(N // blk,),
      in_specs=[
          pl.BlockSpec((blk, DN), lambda i: (i, 0)),
          pl.BlockSpec((blk, DE // 2), lambda i: (i, 0)),
      ],
      out_specs=pl.BlockSpec((blk, DN), lambda i: (i, 0)),
      out_shape=jax.ShapeDtypeStruct((N, DN), jnp.float32),
  )(hp, dp)


def kernel(nfeats, edge_index, efeats, W_fc, W_edge, b_edge, W_coef):
  ei = edge_index.astype(jnp.int32)

  wfc_t = W_fc.T
  ws_t = W_edge[:, :DN].T
  wd_t = W_edge[:, DN + DE:].T
  we_t = W_edge[:, DN:DN + DE].T
  eye8 = jnp.eye(8, dtype=jnp.float32)
  wbd = jnp.kron(eye8, we_t)
  bt = jnp.tile(b_edge, 8)[None, :]
  wcr = jnp.kron(eye8, W_coef[0][:, None] * jnp.ones((1, 16), jnp.float32))

  zl, zr, ps, pd = _tc_node(nfeats, wfc_t, ws_t, wd_t)

  ef8 = efeats.reshape(E8, 128)
  eproj = _tc_edge(ef8, wbd, bt).reshape(E, DE)

  pre_sc, agg_sc = _sc_kernels()
  pre = pre_sc(ps, pd, eproj, ei)

  feat8, ae8 = _tc_att(pre.reshape(E8, 128), wcr)
  feat_e = feat8.reshape(E, DE)
  ae2 = ae8.reshape(E, DE)

  zh = jnp.zeros((NPB, DH), jnp.float32)
  zd = jnp.zeros((NPB, DE // 2), jnp.float32)
  hp, dp = agg_sc(zl, zr, ae2, ei, zh, zd)

  h = _tc_final(hp, dp)
  return (h, feat_e)


# speedup vs baseline: 1.3483x; 1.0280x over previous
import functools

import jax
import jax.numpy as jnp
from jax import lax
from jax.experimental import pallas as pl
from jax.experimental.pallas import tpu as pltpu
from jax.experimental.pallas import tpu_sc as plsc

N = 10000
E = 320000
DN = 128
DE = 16
NC = 2
NS = 16
NW = NC * NS
NPB = N // NS
E8 = E * DE // 128

KA = 5
KAC = KA * 128
SCA = E // KAC
ITA = -(-SCA // NW)
NSLOTA = 5
KB = 2
CH = KB * 128
SCB = E // CH
ITB = -(-SCB // NS)
NSLOT = 3
DH = DN // NC


def _node_body(nf, wfc_t, ws_t, wd_t, zl_out, zr_out, ps_out, pd_out):
  z = jnp.dot(nf[...], wfc_t[...], preferred_element_type=jnp.float32)
  zl_out[...] = z[:, :DH]
  zr_out[...] = z[:, DH:]
  ps_out[...] = jnp.dot(z, ws_t[...], preferred_element_type=jnp.float32)
  pd_out[...] = jnp.dot(z, wd_t[...], preferred_element_type=jnp.float32)


def _edge_body(ef8, wbd, bt, out):
  out[...] = (
      jnp.dot(ef8[...], wbd[...], preferred_element_type=jnp.float32) + bt[...]
  )


def _att_body(g8, ep8, wcr, feat_out, ae_out):
  x = g8[...] + ep8[...]
  f = jnp.where(x >= 0.0, x, 0.01 * x)
  feat_out[...] = f
  ae_out[...] = jnp.exp(
      jnp.dot(f, wcr[...], preferred_element_type=jnp.float32))


def _final_body(p, d, h_out):
  den = jnp.maximum(d[...], 1e-16)
  h_out[...] = p[...] / den[:, 0:1]


def _pre_body(ps_hbm, pd_hbm, ei_hbm, pre_hbm,
              idxs, idxd, buf, semL, semA, semB, semW):
  c = lax.axis_index("c")
  s = lax.axis_index("s")
  w = s * NC + c


  def load_descs(slot, eoff, semref):
    return [
        pltpu.make_async_copy(ei_hbm.at[0, pl.ds(eoff, KAC)],
                              idxs.at[pl.ds(slot * KAC, KAC)], semref),
        pltpu.make_async_copy(ei_hbm.at[1, pl.ds(eoff, KAC)],
                              idxd.at[pl.ds(slot * KAC, KAC)], semref),
    ]

  def gadd_descs(slot, src_hbm, idxref, semref):
    return [
        pltpu.make_async_copy(
            src_hbm.at[idxref.at[pl.ds(slot * KAC + j * 128, 128)]],
            buf.at[pl.ds(slot * KAC + j * 128, 128)], semref)
        for j in range(KA)
    ]

  def wr_desc(slot, eoff, semref):
    return pltpu.make_async_copy(buf.at[pl.ds(slot * KAC, KAC)],
                                 pre_hbm.at[pl.ds(eoff, KAC)], semref)

  def tick(t, r5):
    rW = (r5 - 4) % NSLOTA
    scW = (t - 4) * NW + w

    @pl.when((t >= 4) & (scW < SCA))
    def _():
      wr_desc(rW, scW * KAC, semW.at[rW]).wait()

    scL = t * NW + w

    @pl.when(scL < SCA)
    def _():
      for d in load_descs(r5, scL * KAC, semL.at[r5]):
        d.start()

    rA = (r5 - 1) % NSLOTA
    scA = (t - 1) * NW + w

    @pl.when((t >= 1) & (scA < SCA))
    def _():
      for d in load_descs(rA, scA * KAC, semL.at[rA]):
        d.wait()
      for d in gadd_descs(rA, ps_hbm, idxs, semA.at[rA]):
        d.start()

    rB = (r5 - 2) % NSLOTA
    scB = (t - 2) * NW + w

    @pl.when((t >= 2) & (scB < SCA))
    def _():
      for d in gadd_descs(rB, ps_hbm, idxs, semA.at[rB]):
        d.wait()
      for d in gadd_descs(rB, pd_hbm, idxd, semB.at[rB]):
        d.start(add=True)

    rC = (r5 - 3) % NSLOTA
    scC = (t - 3) * NW + w

    @pl.when((t >= 3) & (scC < SCA))
    def _():
      for d in gadd_descs(rC, pd_hbm, idxd, semB.at[rC]):
        d.wait()
      wr_desc(rC, scC * KAC, semW.at[rC]).start()

  def tickset(i, carry):
    for r5 in range(NSLOTA):
      tick(i * NSLOTA + r5, r5)
    return carry

  nticks = ITA + 4
  lax.fori_loop(0, -(-nticks // NSLOTA), tickset, 0)


def _agg_body(zl_hbm, zr_hbm, ae_hbm, ei_hbm, zh_hbm, zd_hbm,
              hp_hbm, dp_hbm,
              idxs, idxd, rows, avr, avd, hsh, dsh, semL, semG, semS):
  c = lax.axis_index("c")
  s = lax.axis_index("s")

  pltpu.sync_copy(zh_hbm, hsh.at[pl.ds(s * NPB, NPB)])
  pltpu.sync_copy(zd_hbm, dsh.at[pl.ds(s * NPB, NPB)])
  plsc.subcore_barrier()


  def idx_load_descs(slot, eoff, semref):
    ds_ = [
        pltpu.make_async_copy(ei_hbm.at[0, pl.ds(eoff, CH)],
                              idxs.at[pl.ds(slot * CH, CH)], semref),
        pltpu.make_async_copy(ae_hbm.at[pl.ds(eoff, CH)],
                              avr.at[pl.ds(slot * CH, CH)], semref),
        pltpu.make_async_copy(ae_hbm.at[pl.ds(eoff, CH), pl.ds(0, DE // 2)],
                              avd.at[pl.ds(slot * CH, CH)], semref),
    ]
    ds_ += [
        pltpu.make_async_copy(ei_hbm.at[1, pl.ds(eoff + j * 128, 128)],
                              idxd.at[slot * KB + j], semref)
        for j in range(KB)
    ]
    return ds_

  def gather_descs(slot, zsrc, semref):
    return [
        pltpu.make_async_copy(
            zsrc.at[idxs.at[pl.ds(slot * CH + j * 128, 128)]],
            rows.at[pl.ds(slot * CH + j * 128, 128)], semref)
        for j in range(KB)
    ]

  def scat_h_descs(slot, semref):
    return [
        pltpu.make_async_copy(rows.at[pl.ds(slot * CH + j * 128, 128)],
                              hsh.at[idxd.at[slot * KB + j]], semref)
        for j in range(KB)
    ]

  def scat_d_descs(slot, semref):
    return [
        pltpu.make_async_copy(avd.at[pl.ds(slot * CH + j * 128, 128)],
                              dsh.at[idxd.at[slot * KB + j]], semref)
        for j in range(KB)
    ]

  def tick(t, r4):
    rD = (r4 - 3) % NSLOT
    scD = (t - 3) * NS + s

    @pl.when((t >= 3) & (scD < SCB))
    def _():
      for d in scat_h_descs(rD, semS.at[rD]):
        d.wait()

      @pl.when(c == 0)
      def _():
        for d in scat_d_descs(rD, semS.at[rD]):
          d.wait()

    scL = t * NS + s

    @pl.when(scL < SCB)
    def _():
      for d in idx_load_descs(r4, scL * CH, semL.at[r4]):
        d.start()

    rG = (r4 - 1) % NSLOT
    scG = (t - 1) * NS + s

    @pl.when((t >= 1) & (scG < SCB))
    def _():
      for d in idx_load_descs(rG, scG * CH, semL.at[rG]):
        d.wait()

      @pl.when(c == 0)
      def _():
        for d in gather_descs(rG, zl_hbm, semG.at[rG]):
          d.start()

      @pl.when(c == 1)
      def _():
        for d in gather_descs(rG, zr_hbm, semG.at[rG]):
          d.start()

    rC = (r4 - 2) % NSLOT
    scC = (t - 2) * NS + s

    @pl.when((t >= 2) & (scC < SCB))
    def _():
      @pl.when(c == 0)
      def _():
        for d in gather_descs(rC, zl_hbm, semG.at[rC]):
          d.wait()

      @pl.when(c == 1)
      def _():
        for d in gather_descs(rC, zr_hbm, semG.at[rC]):
          d.wait()

      def scale_grp(g, carry2):
        for l in range(16):
          r = rC * CH + g * 16 + l
          va = avr[r]
          for k in range(DH // 16):
            sl = pl.ds(k * 16, 16)
            rows[r, sl] = rows[r, sl] * va
        return carry2

      lax.fori_loop(0, KB * 8, scale_grp, 0)

      for d in scat_h_descs(rC, semS.at[rC]):
        d.start(add=True)

      @pl.when(c == 0)
      def _():
        for d in scat_d_descs(rC, semS.at[rC]):
          d.start(add=True)

  def tickset(i, carry):
    for r4 in range(NSLOT):
      tick(i * NSLOT + r4, r4)
    return carry

  nticks = ITB + 3
  lax.fori_loop(0, -(-nticks // NSLOT), tickset, 0)
  plsc.subcore_barrier()

  @pl.when(c == 0)
  def _():
    pltpu.sync_copy(hsh.at[pl.ds(s * NPB, NPB)],
                    hp_hbm.at[pl.ds(s * NPB, NPB), pl.ds(0, DH)])
    pltpu.sync_copy(dsh.at[pl.ds(s * NPB, NPB)],
                    dp_hbm.at[pl.ds(s * NPB, NPB)])

  @pl.when(c == 1)
  def _():
    pltpu.sync_copy(hsh.at[pl.ds(s * NPB, NPB)],
                    hp_hbm.at[pl.ds(s * NPB, NPB), pl.ds(DH, DH)])


@functools.lru_cache(maxsize=None)
def _sc_kernels():
  mesh = plsc.VectorSubcoreMesh(
      core_axis_name="c", subcore_axis_name="s", num_cores=NC, num_subcores=NS
  )
  params = pltpu.CompilerParams(use_tc_tiling_on_sc=False)
  pre_sc = pl.kernel(
      _pre_body,
      out_type=jax.ShapeDtypeStruct((E, DE), jnp.float32),
      mesh=mesh,
      compiler_params=params,
      scratch_types=[
          pltpu.VMEM((NSLOTA * KAC,), jnp.int32),
          pltpu.VMEM((NSLOTA * KAC,), jnp.int32),
          pltpu.VMEM((NSLOTA * KAC, DE), jnp.float32),
          pltpu.SemaphoreType.DMA((NSLOTA,)),
          pltpu.SemaphoreType.DMA((NSLOTA,)),
          pltpu.SemaphoreType.DMA((NSLOTA,)),
          pltpu.SemaphoreType.DMA((NSLOTA,)),
      ],
  )
  agg_sc = pl.kernel(
      _agg_body,
      out_type=(
          jax.ShapeDtypeStruct((N, DN), jnp.float32),
          jax.ShapeDtypeStruct((N, DE // 2), jnp.float32),
      ),
      mesh=mesh,
      compiler_params=params,
      scratch_types=[
          pltpu.VMEM((NSLOT * CH,), jnp.int32),
          pltpu.VMEM((NSLOT * KB, 128), jnp.int32),
          pltpu.VMEM((NSLOT * CH, DH), jnp.float32),
          pltpu.VMEM((NSLOT * CH, DE), jnp.float32),
          pltpu.VMEM((NSLOT * CH, DE // 2), jnp.float32),
          pltpu.VMEM_SHARED((N, DH), jnp.float32),
          pltpu.VMEM_SHARED((N, DE // 2), jnp.float32),
          pltpu.SemaphoreType.DMA((NSLOT,)),
          pltpu.SemaphoreType.DMA((NSLOT,)),
          pltpu.SemaphoreType.DMA((NSLOT,)),
      ],
  )
  return pre_sc, agg_sc


def _tc_node(nfeats, wfc_t, ws_t, wd_t):
  blk = 1000
  return pl.pallas_call(
      _node_body,
      grid=(N // blk,),
      in_specs=[
          pl.BlockSpec((blk, DN), lambda i: (i, 0)),
          pl.BlockSpec((DN, DN), lambda i: (0, 0)),
          pl.BlockSpec((DN, DE), lambda i: (0, 0)),
          pl.BlockSpec((DN, DE), lambda i: (0, 0)),
      ],
      out_specs=[
          pl.BlockSpec((blk, DH), lambda i: (i, 0)),
          pl.BlockSpec((blk, DH), lambda i: (i, 0)),
          pl.BlockSpec((blk, DE), lambda i: (i, 0)),
          pl.BlockSpec((blk, DE), lambda i: (i, 0)),
      ],
      out_shape=[
          jax.ShapeDtypeStruct((N, DH), jnp.float32),
          jax.ShapeDtypeStruct((N, DH), jnp.float32),
          jax.ShapeDtypeStruct((N, DE), jnp.float32),
          jax.ShapeDtypeStruct((N, DE), jnp.float32),
      ],
  )(nfeats, wfc_t, ws_t, wd_t)


def _tc_edge(ef8, wbd, bt):
  blk = E8 // 10
  return pl.pallas_call(
      _edge_body,
      grid=(10,),
      in_specs=[
          pl.BlockSpec((blk, 128), lambda i: (i, 0)),
          pl.BlockSpec((128, 128), lambda i: (0, 0)),
          pl.BlockSpec((1, 128), lambda i: (0, 0)),
      ],
      out_specs=pl.BlockSpec((blk, 128), lambda i: (i, 0)),
      out_shape=jax.ShapeDtypeStruct((E8, 128), jnp.float32),
  )(ef8, wbd, bt)


def _tc_att(g8, ep8, wcr):
  blk8 = E8 // 10
  return pl.pallas_call(
      _att_body,
      grid=(10,),
      in_specs=[
          pl.BlockSpec((blk8, 128), lambda i: (i, 0)),
          pl.BlockSpec((blk8, 128), lambda i: (i, 0)),
          pl.BlockSpec((128, 128), lambda i: (0, 0)),
      ],
      out_specs=[
          pl.BlockSpec((blk8, 128), lambda i: (i, 0)),
          pl.BlockSpec((blk8, 128), lambda i: (i, 0)),
      ],
      out_shape=[
          jax.ShapeDtypeStruct((E8, 128), jnp.float32),
          jax.ShapeDtypeStruct((E8, 128), jnp.float32),
      ],
  )(g8, ep8, wcr)


def _tc_final(hp, dp):
  blk = 1000
  return pl.pallas_call(
      _final_body,
      grid=(N // blk,),
      in_specs=[
          pl.BlockSpec((blk, DN), lambda i: (i, 0)),
          pl.BlockSpec((blk, DE // 2), lambda i: (i, 0)),
      ],
      out_specs=pl.BlockSpec((blk, DN), lambda i: (i, 0)),
      out_shape=jax.ShapeDtypeStruct((N, DN), jnp.float32),
  )(hp, dp)


def kernel(nfeats, edge_index, efeats, W_fc, W_edge, b_edge, W_coef):
  ei = edge_index.astype(jnp.int32)

  wfc_t = W_fc.T
  ws_t = W_edge[:, :DN].T
  wd_t = W_edge[:, DN + DE:].T
  we_t = W_edge[:, DN:DN + DE].T
  eye8 = jnp.eye(8, dtype=jnp.float32)
  wbd = jnp.kron(eye8, we_t)
  bt = jnp.tile(b_edge, 8)[None, :]
  wcr = jnp.kron(eye8, W_coef[0][:, None] * jnp.ones((1, 16), jnp.float32))

  zl, zr, ps, pd = _tc_node(nfeats, wfc_t, ws_t, wd_t)

  ef8 = efeats.reshape(E8, 128)
  eproj8 = _tc_edge(ef8, wbd, bt)

  pre_sc, agg_sc = _sc_kernels()
  g = pre_sc(ps, pd, ei)

  feat8, ae8 = _tc_att(g.reshape(E8, 128), eproj8, wcr)
  feat_e = feat8.reshape(E, DE)
  ae2 = ae8.reshape(E, DE)

  zh = jnp.zeros((NPB, DH), jnp.float32)
  zd = jnp.zeros((NPB, DE // 2), jnp.float32)
  hp, dp = agg_sc(zl, zr, ae2, ei, zh, zd)

  h = _tc_final(hp, dp)
  return (h, feat_e)
